# speedup vs baseline: 5.9275x; 5.9275x over previous
import functools

import jax
import jax.numpy as jnp
from jax import lax
from jax.experimental import pallas as pl
from jax.experimental.pallas import tpu as pltpu
from jax.experimental.pallas import tpu_sc as plsc

_NC = 2
_NS = 16
_NW = _NC * _NS

_C = 128


@functools.partial(jax.jit, static_argnums=(2, 3))
def _gather_flat(idx, table, b, d):
    b_per_w = b // _NW
    n_chunks = b_per_w // _C

    mesh = plsc.VectorSubcoreMesh(core_axis_name="c", subcore_axis_name="s")

    @functools.partial(
        pl.kernel,
        mesh=mesh,
        out_type=jax.ShapeDtypeStruct((b, d), jnp.float32),
        scratch_types=[
            pltpu.VMEM((_C,), jnp.int32),
            pltpu.VMEM((_C, d), jnp.float32),
            pltpu.SemaphoreType.DMA,
        ],
    )
    def k(idx_hbm, table_hbm, out_hbm, idx_v, rows_v, sem):
        wid = lax.axis_index("s") * _NC + lax.axis_index("c")
        base = wid * b_per_w

        def body(i, carry):
            off = pl.multiple_of(base + i * _C, _C)
            pltpu.sync_copy(idx_hbm.at[pl.ds(off, _C)], idx_v)
            pltpu.async_copy(table_hbm.at[idx_v], rows_v, sem).wait()
            pltpu.sync_copy(rows_v, out_hbm.at[pl.ds(off, _C)])
            return carry

        lax.fori_loop(0, n_chunks, body, 0)

    return k(idx, table)


def kernel(x, table):
    b = x.size
    d = table.shape[1]
    idx = x.reshape((b,)).astype(jnp.int32)
    out = _gather_flat(idx, table, b, d)
    return lax.stop_gradient(out.reshape(x.shape + (d,)))


# speedup vs baseline: 8.5776x; 1.4471x over previous
import functools

import jax
import jax.numpy as jnp
from jax import lax
from jax.experimental import pallas as pl
from jax.experimental.pallas import tpu as pltpu
from jax.experimental.pallas import tpu_sc as plsc

_NC = 2
_NS = 16
_NW = _NC * _NS

_C = 128
_S = 8
_R = 4


@functools.partial(jax.jit, static_argnums=(2, 3))
def _gather_flat(idx2d, table, b, d):
    b_per_w = b // _NW
    n_chunks = b_per_w // _C
    n_super = n_chunks // _S

    mesh = plsc.VectorSubcoreMesh(core_axis_name="c", subcore_axis_name="s")

    @functools.partial(
        pl.kernel,
        mesh=mesh,
        out_type=jax.ShapeDtypeStruct((b, d), jnp.float32),
        scratch_types=[
            pltpu.VMEM((_S, _C), jnp.int32),
            pltpu.VMEM((_R, _C, d), jnp.float32),
        ]
        + [pltpu.SemaphoreType.DMA] * (2 * _R),
    )
    def k(idx_hbm, table_hbm, out_hbm, idx_v, rows, *sems):
        gs = sems[:_R]
        os_ = sems[_R:]

        wid = lax.axis_index("s") * _NC + lax.axis_index("c")
        crow0 = wid * n_chunks

        def fire_gather(j, g):
            pltpu.async_copy(
                table_hbm.at[idx_v.at[j]], rows.at[j % _R], gs[j % _R]
            )

        def wait_gather(s):
            pltpu.make_async_copy(
                out_hbm.at[pl.ds(0, _C)], rows.at[s], gs[s]
            ).wait()

        def fire_out(s, crow):
            off = pl.multiple_of(crow * _C, _C)
            pltpu.async_copy(rows.at[s], out_hbm.at[pl.ds(off, _C)], os_[s])

        def wait_out(s):
            pltpu.make_async_copy(
                rows.at[s], out_hbm.at[pl.ds(0, _C)], os_[s]
            ).wait()

        pltpu.sync_copy(idx_hbm.at[pl.ds(crow0, _S)], idx_v)
        fire_gather(0, 0)
        for j in range(1, _S):
            s, ps = j % _R, (j - 1) % _R
            wait_gather(ps)
            fire_out(ps, crow0 + j - 1)
            if j >= _R:
                wait_out(s)
            fire_gather(j, 0)

        def body(g, carry):
            crow = crow0 + g * _S
            wait_gather((_S - 1) % _R)
            fire_out((_S - 1) % _R, crow - 1)
            pltpu.sync_copy(idx_hbm.at[pl.ds(crow, _S)], idx_v)
            for j in range(_S):
                s = j % _R
                if j > 0:
                    wait_gather((j - 1) % _R)
                    fire_out((j - 1) % _R, crow + j - 1)
                wait_out(s)
                fire_gather(j, g)
            return carry

        lax.fori_loop(1, n_super, body, 0)

        last = crow0 + n_chunks - 1
        wait_gather((_S - 1) % _R)
        fire_out((_S - 1) % _R, last)
        for s in range(_R):
            wait_out(s)

    return k(idx2d, table)


def kernel(x, table):
    b = x.size
    d = table.shape[1]
    idx2d = x.reshape((b // _C, _C)).astype(jnp.int32)
    out = _gather_flat(idx2d, table, b, d)
    return lax.stop_gradient(out.reshape(x.shape + (d,)))


# speedup vs baseline: 8.8131x; 1.0275x over previous
import functools

import jax
import jax.numpy as jnp
from jax import lax
from jax.experimental import pallas as pl
from jax.experimental.pallas import tpu as pltpu
from jax.experimental.pallas import tpu_sc as plsc

_NC = 2
_NS = 16
_NW = _NC * _NS

_C = 128
_S = 8
_R = 4


@functools.partial(jax.jit, static_argnums=(2, 3))
def _gather_flat(idx2d, table, b, d):
    b_per_w = b // _NW
    n_chunks = b_per_w // _C
    n_super = n_chunks // _S

    mesh = plsc.VectorSubcoreMesh(core_axis_name="c", subcore_axis_name="s")

    @functools.partial(
        pl.kernel,
        mesh=mesh,
        out_type=jax.ShapeDtypeStruct((b, d), jnp.float32),
        scratch_types=[
            pltpu.VMEM((2, _S, _C), jnp.int32),
            pltpu.VMEM((_R, _C, d), jnp.float32),
        ]
        + [pltpu.SemaphoreType.DMA] * (2 * _R + 2),
    )
    def k(idx_hbm, table_hbm, out_hbm, idx_v, rows, *sems):
        gs = sems[:_R]
        os_ = sems[_R:2 * _R]
        is_ = sems[2 * _R:]

        wid = lax.axis_index("s") * _NC + lax.axis_index("c")
        crow0 = wid * n_chunks

        def fire_idx(g, p):
            pltpu.async_copy(idx_hbm.at[pl.ds(crow0 + g * _S, _S)],
                             idx_v.at[p], is_[p])

        def wait_idx(p):
            pltpu.make_async_copy(idx_hbm.at[pl.ds(0, _S)], idx_v.at[p],
                                  is_[p]).wait()

        def fire_gather(p, j):
            pltpu.async_copy(table_hbm.at[idx_v.at[p, j]], rows.at[j % _R],
                             gs[j % _R])

        def wait_gather(s):
            pltpu.make_async_copy(out_hbm.at[pl.ds(0, _C)], rows.at[s],
                                  gs[s]).wait()

        def fire_out(s, crow):
            off = pl.multiple_of(crow * _C, _C)
            pltpu.async_copy(rows.at[s], out_hbm.at[pl.ds(off, _C)], os_[s])

        def wait_out(s):
            pltpu.make_async_copy(rows.at[s], out_hbm.at[pl.ds(0, _C)],
                                  os_[s]).wait()

        def superblock(p, crow, first):
            for j in range(_S):
                s = j % _R
                if j > 0:
                    wait_gather((j - 1) % _R)
                    fire_out((j - 1) % _R, crow + j - 1)
                if not (first and j < _R):
                    wait_out(s)
                fire_gather(p, j)

        fire_idx(0, 0)
        fire_idx(1, 1)
        wait_idx(0)
        superblock(0, crow0, first=True)

        wait_gather((_S - 1) % _R)
        fire_out((_S - 1) % _R, crow0 + _S - 1)
        wait_idx(1)
        fire_idx(2, 0)
        superblock(1, crow0 + _S, first=False)

        def body(t, carry):
            g0 = 2 * t
            crow = crow0 + g0 * _S
            wait_gather((_S - 1) % _R)
            fire_out((_S - 1) % _R, crow - 1)
            wait_idx(0)
            fire_idx(jnp.minimum(g0 + 1, n_super - 1), 1)
            superblock(0, crow, first=False)
            wait_gather((_S - 1) % _R)
            fire_out((_S - 1) % _R, crow + _S - 1)
            wait_idx(1)
            fire_idx(jnp.minimum(g0 + 2, n_super - 1), 0)
            superblock(1, crow + _S, first=False)
            return carry

        lax.fori_loop(1, n_super // 2, body, 0)

        wait_gather((_S - 1) % _R)
        fire_out((_S - 1) % _R, crow0 + n_chunks - 1)
        wait_idx(0)
        for s in range(_R):
            wait_out(s)

    return k(idx2d, table)


def kernel(x, table):
    b = x.size
    d = table.shape[1]
    idx2d = x.reshape((b // _C, _C)).astype(jnp.int32)
    out = _gather_flat(idx2d, table, b, d)
    return lax.stop_gradient(out.reshape(x.shape + (d,)))


# speedup vs baseline: 10.8357x; 1.2295x over previous
import functools

import jax
import jax.numpy as jnp
from jax import lax
from jax.experimental import pallas as pl
from jax.experimental.pallas import tpu as pltpu
from jax.experimental.pallas import tpu_sc as plsc

_NC = 2
_NS = 16
_NW = _NC * _NS

_C = 128
_S = 8
_R = 4
_G = 2


@functools.partial(jax.jit, static_argnums=(2, 3))
def _gather_flat(idx2d, table, b, d):
    b_per_w = b // _NW
    n_chunks = b_per_w // _C
    n_super = n_chunks // _S

    mesh = plsc.VectorSubcoreMesh(core_axis_name="c", subcore_axis_name="s")

    @functools.partial(
        pl.kernel,
        mesh=mesh,
        out_type=jax.ShapeDtypeStruct((b, d), jnp.float32),
        scratch_types=[
            pltpu.VMEM((2, _S, _C), jnp.int32),
            pltpu.VMEM((_R, _C, d), jnp.float32),
        ]
        + [pltpu.SemaphoreType.DMA] * (2 * _R + 2),
    )
    def k(idx_hbm, table_hbm, out_hbm, idx_v, rows, *sems):
        gs = sems[:_R]
        os_ = sems[_R:2 * _R]
        is_ = sems[2 * _R:]

        wid = lax.axis_index("s") * _NC + lax.axis_index("c")
        crow0 = wid * n_chunks

        def fire_idx(g, p):
            pltpu.async_copy(idx_hbm.at[pl.ds(crow0 + g * _S, _S)],
                             idx_v.at[p], is_[p])

        def wait_idx(p):
            pltpu.make_async_copy(idx_hbm.at[pl.ds(0, _S)], idx_v.at[p],
                                  is_[p]).wait()

        def fire_gather(p, j):
            pltpu.async_copy(table_hbm.at[idx_v.at[p, j]], rows.at[j % _R],
                             gs[j % _R])

        def wait_gather(s):
            pltpu.make_async_copy(out_hbm.at[pl.ds(0, _C)], rows.at[s],
                                  gs[s]).wait()

        def fire_out(s, crow):
            off = pl.multiple_of(crow * _C, _C)
            pltpu.async_copy(rows.at[s], out_hbm.at[pl.ds(off, _C)], os_[s])

        def wait_out(s):
            pltpu.make_async_copy(rows.at[s], out_hbm.at[pl.ds(0, _C)],
                                  os_[s]).wait()

        def superblock(p, crow, first, next_load=None):
            for j in range(_S):
                s = j % _R
                if not (first and j < _R):
                    wait_out(s)
                fire_gather(p, j)
                if not (first and j < _G):
                    ps = (j - _G) % _R
                    wait_gather(ps)
                    fire_out(ps, crow + j - _G)
                if j == _G - 1 and next_load is not None:
                    fire_idx(*next_load)

        fire_idx(0, 0)
        fire_idx(1, 1)
        wait_idx(0)
        superblock(0, crow0, first=True)
        wait_idx(1)
        superblock(1, crow0 + _S, first=False, next_load=(2, 0))

        def body(t, carry):
            g0 = 2 * t
            crow = crow0 + g0 * _S
            wait_idx(0)
            superblock(0, crow, first=False,
                       next_load=(jnp.minimum(g0 + 1, n_super - 1), 1))
            wait_idx(1)
            superblock(1, crow + _S, first=False,
                       next_load=(jnp.minimum(g0 + 2, n_super - 1), 0))
            return carry

        lax.fori_loop(1, n_super // 2, body, 0)

        for j in range(_G):
            ps = (_S - _G + j) % _R
            wait_gather(ps)
            fire_out(ps, crow0 + n_chunks - _G + j)
        wait_idx(0)
        for s in range(_R):
            wait_out(s)

    return k(idx2d, table)


def kernel(x, table):
    b = x.size
    d = table.shape[1]
    idx2d = x.reshape((b // _C, _C)).astype(jnp.int32)
    out = _gather_flat(idx2d, table, b, d)
    return lax.stop_gradient(out.reshape(x.shape + (d,)))
